# baseline (speedup 1.0000x reference)
import functools

import numpy as np
import jax
import jax.numpy as jnp
from jax import lax
from jax.experimental import pallas as pl
from jax.experimental.pallas import tpu as pltpu
from jax.experimental.pallas import tpu_sc as plsc

_PERCENT = 0.1


@functools.lru_cache(maxsize=None)
def _steps_and_bias(B, T, n_idx):
    candidates = jnp.array([-1.0, -0.5, 0.5, 1.0], dtype=jnp.float32)
    kk = jax.random.key(42)
    ks, kb = jax.random.split(kk)
    n_steps = int(T * _PERCENT)
    steps = jax.random.permutation(ks, T)[:n_steps]
    bias = jax.random.choice(kb, candidates, shape=(B, n_steps, n_idx))
    return tuple(int(s) for s in np.asarray(steps)), np.asarray(bias)


def _make_sc_kernel(B, T, C, n_steps, n_idx, steps):
    info = plsc.get_sparse_core_info()
    NC, NS, L = info.num_cores, info.num_subcores, info.num_lanes
    NW = NC * NS
    TC = T * C
    BW = n_steps * n_idx
    assert B % NW == 0 and n_idx % L == 0
    rows_per = B // NW
    R = 4
    assert rows_per % R == 0
    n_chunks = rows_per // R
    mesh = plsc.VectorSubcoreMesh(core_axis_name="c", subcore_axis_name="s")

    @functools.partial(
        pl.kernel, mesh=mesh,
        compiler_params=pltpu.CompilerParams(needs_layout_passes=False),
        out_type=jax.ShapeDtypeStruct((B * TC,), jnp.float32),
        scratch_types=[
            pltpu.VMEM((R * TC,), jnp.float32),
            pltpu.VMEM((R * BW,), jnp.float32),
            pltpu.VMEM((n_idx,), jnp.int32),
        ],
    )
    def k(ts_hbm, bias_hbm, idx_hbm, out_hbm, buf_v, bias_v, idx_v):
        wid = lax.axis_index("s") * NC + lax.axis_index("c")
        pltpu.sync_copy(idx_hbm, idx_v)
        idx_parts = [idx_v[pl.ds(p * L, L)] for p in range(n_idx // L)]
        base_row = wid * rows_per

        def chunk_body(c, carry):
            row0 = base_row + c * R
            pltpu.sync_copy(ts_hbm.at[pl.ds(row0 * TC, R * TC)], buf_v)
            pltpu.sync_copy(bias_hbm.at[pl.ds(row0 * BW, R * BW)], bias_v)
            for r in range(R):
                for i, s in enumerate(steps):
                    for p in range(n_idx // L):
                        tgt = idx_parts[p] + jnp.int32(r * TC + s * C)
                        vals = bias_v[pl.ds(r * BW + i * n_idx + p * L, L)]
                        plsc.addupdate_scatter(buf_v, [tgt], vals)
            pltpu.sync_copy(buf_v, out_hbm.at[pl.ds(row0 * TC, R * TC)])
            return carry

        lax.fori_loop(0, n_chunks, chunk_body, 0)

    return k


def kernel(ts, indices):
    B, T, C = ts.shape
    n_idx = indices.shape[0]
    with jax.ensure_compile_time_eval():
        steps, bias = _steps_and_bias(B, T, n_idx)
    n_steps = len(steps)
    k = _make_sc_kernel(B, T, C, n_steps, n_idx, steps)
    out = k(ts.reshape(B * T * C),
            jnp.asarray(bias).reshape(B * n_steps * n_idx),
            indices.astype(jnp.int32))
    return out.reshape(B, T, C)


# speedup vs baseline: 1.0406x; 1.0406x over previous
import functools

import numpy as np
import jax
import jax.numpy as jnp
from jax import lax
from jax.experimental import pallas as pl
from jax.experimental.pallas import tpu as pltpu
from jax.experimental.pallas import tpu_sc as plsc

_PERCENT = 0.1


@functools.lru_cache(maxsize=None)
def _steps_and_bias(B, T, n_idx):
    candidates = jnp.array([-1.0, -0.5, 0.5, 1.0], dtype=jnp.float32)
    kk = jax.random.key(42)
    ks, kb = jax.random.split(kk)
    n_steps = int(T * _PERCENT)
    steps = jax.random.permutation(ks, T)[:n_steps]
    bias = jax.random.choice(kb, candidates, shape=(B, n_steps, n_idx))
    return tuple(int(s) for s in np.asarray(steps)), np.asarray(bias)


def _make_sc_kernel(B, T, C, n_steps, n_idx, steps):
    info = plsc.get_sparse_core_info()
    NC, NS, L = info.num_cores, info.num_subcores, info.num_lanes
    NW = NC * NS
    TC = T * C
    BW = n_steps * n_idx
    assert B % NW == 0 and n_idx % L == 0
    rows_per = B // NW
    R = 2
    NBUF = 4
    assert rows_per % R == 0
    n_chunks = rows_per // R
    mesh = plsc.VectorSubcoreMesh(core_axis_name="c", subcore_axis_name="s")

    @functools.partial(
        pl.kernel, mesh=mesh,
        compiler_params=pltpu.CompilerParams(needs_layout_passes=False),
        out_type=jax.ShapeDtypeStruct((B * TC,), jnp.float32),
        scratch_types=(
            [pltpu.VMEM((R * TC,), jnp.float32)] * NBUF
            + [pltpu.VMEM((R * BW,), jnp.float32)] * NBUF
            + [pltpu.VMEM((n_idx,), jnp.int32)]
            + [pltpu.SemaphoreType.DMA] * (2 * NBUF)
        ),
    )
    def k(ts_hbm, bias_hbm, idx_hbm, out_hbm, *refs):
        bufs = refs[:NBUF]
        bbufs = refs[NBUF:2 * NBUF]
        idx_v = refs[2 * NBUF]
        sin = refs[2 * NBUF + 1:2 * NBUF + 1 + NBUF]
        sout = refs[2 * NBUF + 1 + NBUF:]
        wid = lax.axis_index("s") * NC + lax.axis_index("c")
        pltpu.sync_copy(idx_hbm, idx_v)
        idx_parts = [idx_v[pl.ds(p * L, L)] for p in range(n_idx // L)]
        base_row = wid * rows_per

        in_h, out_h = {}, {}

        def start_in(c):
            sl = c % NBUF
            row0 = base_row + c * R
            in_h[c] = (
                pltpu.async_copy(ts_hbm.at[pl.ds(row0 * TC, R * TC)],
                                 bufs[sl], sin[sl]),
                pltpu.async_copy(bias_hbm.at[pl.ds(row0 * BW, R * BW)],
                                 bbufs[sl], sin[sl]),
            )

        for c in range(min(NBUF - 1, n_chunks)):
            start_in(c)

        for c in range(n_chunks):
            sl = c % NBUF
            for h in in_h.pop(c):
                h.wait()
            for r in range(R):
                for i, s in enumerate(steps):
                    for p in range(n_idx // L):
                        tgt = idx_parts[p] + jnp.int32(r * TC + s * C)
                        vals = bbufs[sl][pl.ds(r * BW + i * n_idx + p * L, L)]
                        plsc.addupdate_scatter(bufs[sl], [tgt], vals)
            row0 = base_row + c * R
            out_h[c] = pltpu.async_copy(
                bufs[sl], out_hbm.at[pl.ds(row0 * TC, R * TC)], sout[sl])
            nxt = c + NBUF - 1
            if nxt < n_chunks:
                if nxt >= NBUF:
                    out_h.pop(nxt - NBUF).wait()
                start_in(nxt)

        for c in sorted(out_h):
            out_h.pop(c).wait()

    return k


def kernel(ts, indices):
    B, T, C = ts.shape
    n_idx = indices.shape[0]
    with jax.ensure_compile_time_eval():
        steps, bias = _steps_and_bias(B, T, n_idx)
    n_steps = len(steps)
    k = _make_sc_kernel(B, T, C, n_steps, n_idx, steps)
    out = k(ts.reshape(B * T * C),
            jnp.asarray(bias).reshape(B * n_steps * n_idx),
            indices.astype(jnp.int32))
    return out.reshape(B, T, C)


# speedup vs baseline: 2.0915x; 2.0098x over previous
import functools

import numpy as np
import jax
import jax.numpy as jnp
from jax import lax
from jax.experimental import pallas as pl
from jax.experimental.pallas import tpu as pltpu
from jax.experimental.pallas import tpu_sc as plsc

_PERCENT = 0.1


@functools.lru_cache(maxsize=None)
def _steps_and_bias(B, T, n_idx):
    candidates = jnp.array([-1.0, -0.5, 0.5, 1.0], dtype=jnp.float32)
    kk = jax.random.key(42)
    ks, kb = jax.random.split(kk)
    n_steps = int(T * _PERCENT)
    steps = jax.random.permutation(ks, T)[:n_steps]
    bias = jax.random.choice(kb, candidates, shape=(B, n_steps, n_idx))
    return tuple(int(s) for s in np.asarray(steps)), np.asarray(bias)


def _make_sc_kernel(B, T, C, n_steps, n_idx, steps):
    info = plsc.get_sparse_core_info()
    NC, NS, L = info.num_cores, info.num_subcores, info.num_lanes
    NW = NC * NS
    TC = T * C
    BW = n_steps * n_idx
    assert B % NW == 0 and n_idx % L == 0
    rows_per = B // NW
    R = 2
    NBUF = 4
    assert rows_per % R == 0
    n_chunks = rows_per // R
    mesh = plsc.VectorSubcoreMesh(core_axis_name="c", subcore_axis_name="s")

    @functools.partial(
        pl.kernel, mesh=mesh,
        compiler_params=pltpu.CompilerParams(needs_layout_passes=False),
        out_type=jax.ShapeDtypeStruct((B, TC), jnp.float32),
        scratch_types=(
            [pltpu.VMEM((R, TC), jnp.float32)] * NBUF
            + [pltpu.VMEM((R * BW,), jnp.float32)] * NBUF
            + [pltpu.VMEM((n_idx,), jnp.int32)]
            + [pltpu.SemaphoreType.DMA] * (2 * NBUF)
        ),
    )
    def k(ts_hbm, bias_hbm, idx_hbm, out_hbm, *refs):
        bufs = refs[:NBUF]
        bbufs = refs[NBUF:2 * NBUF]
        idx_v = refs[2 * NBUF]
        sin = refs[2 * NBUF + 1:2 * NBUF + 1 + NBUF]
        sout = refs[2 * NBUF + 1 + NBUF:]
        wid = lax.axis_index("s") * NC + lax.axis_index("c")
        pltpu.sync_copy(idx_hbm, idx_v)
        idx_parts = [idx_v[pl.ds(p * L, L)] for p in range(n_idx // L)]
        base_row = wid * rows_per

        in_h, out_h = {}, {}

        def start_in(c):
            sl = c % NBUF
            row0 = base_row + c * R
            in_h[c] = (
                pltpu.async_copy(ts_hbm.at[pl.ds(row0, R)],
                                 bufs[sl], sin[sl]),
                pltpu.async_copy(bias_hbm.at[pl.ds(row0 * BW, R * BW)],
                                 bbufs[sl], sin[sl]),
            )

        for c in range(min(NBUF - 1, n_chunks)):
            start_in(c)

        for c in range(n_chunks):
            sl = c % NBUF
            for h in in_h.pop(c):
                h.wait()
            for r in range(R):
                row_splat = jnp.full((L,), r, dtype=jnp.int32)
                for i, s in enumerate(steps):
                    for p in range(n_idx // L):
                        tgt = idx_parts[p] + jnp.int32(s * C)
                        vals = bbufs[sl][pl.ds(r * BW + i * n_idx + p * L, L)]
                        plsc.addupdate_scatter(bufs[sl], [row_splat, tgt], vals)
            row0 = base_row + c * R
            out_h[c] = pltpu.async_copy(
                bufs[sl], out_hbm.at[pl.ds(row0, R)], sout[sl])
            nxt = c + NBUF - 1
            if nxt < n_chunks:
                if nxt >= NBUF:
                    out_h.pop(nxt - NBUF).wait()
                start_in(nxt)

        for c in sorted(out_h):
            out_h.pop(c).wait()

    return k


def kernel(ts, indices):
    B, T, C = ts.shape
    n_idx = indices.shape[0]
    with jax.ensure_compile_time_eval():
        steps, bias = _steps_and_bias(B, T, n_idx)
    n_steps = len(steps)
    k = _make_sc_kernel(B, T, C, n_steps, n_idx, steps)
    out = k(ts.reshape(B, T * C),
            jnp.asarray(bias).reshape(B * n_steps * n_idx),
            indices.astype(jnp.int32))
    return out.reshape(B, T, C)
